# Initial kernel scaffold; baseline (speedup 1.0000x reference)
import jax
import jax.numpy as jnp
from jax.experimental import pallas as pl


def kernel(z, embedding, ema_cluster_size, ema_w):
    raise NotImplementedError("write your pallas kernel here")


# speedup vs baseline: 3.2753x; 3.2753x over previous
import jax
import jax.numpy as jnp
from jax.experimental import pallas as pl
from jax.experimental.pallas import tpu as pltpu

_STATE = 8
_DICT = 1024
_EMB = 32
_BATCH = 512


def _vq_kernel(z_ref, emb_ref, zq_ref, idx_ref, loss_ref):
    s = pl.program_id(0)
    z = z_ref[0]
    e = emb_ref[0]
    en = jnp.sum(e * e, axis=1, keepdims=True)
    z_aug = jnp.concatenate(
        [z, jnp.ones((_BATCH, 1), jnp.float32),
         jnp.zeros((_BATCH, 128 - _EMB - 1), jnp.float32)], axis=1)
    e_aug = jnp.concatenate(
        [-2.0 * e, en,
         jnp.zeros((_DICT, 128 - _EMB - 1), jnp.float32)], axis=1)
    score = jax.lax.dot_general(
        z_aug, e_aug, (((1,), (1,)), ((), ())),
        preferred_element_type=jnp.float32,
        precision=jax.lax.Precision.HIGHEST)
    mins = jnp.min(score, axis=1, keepdims=True)
    kiota = jax.lax.broadcasted_iota(jnp.int32, (_BATCH, _DICT), 1)
    idx = jnp.min(jnp.where(score == mins, kiota, _DICT), axis=1).astype(jnp.int32)
    onehot = (kiota == idx[:, None]).astype(jnp.float32)
    zq = jax.lax.dot_general(
        onehot, e, (((1,), (0,)), ((), ())),
        preferred_element_type=jnp.float32,
        precision=jax.lax.Precision.HIGHEST)
    diff = zq - z
    zq_ref[0] = z + diff
    idx_ref[0, 0] = idx

    @pl.when(s == 0)
    def _init():
        loss_ref[...] = jnp.zeros((1, 1), jnp.float32)

    loss_ref[...] += jnp.sum(diff * diff).reshape(1, 1)

    @pl.when(s == _STATE - 1)
    def _finalize():
        loss_ref[...] = loss_ref[...] / float(_BATCH * _STATE * _EMB)


def kernel(z, embedding, ema_cluster_size, ema_w):
    del ema_cluster_size, ema_w
    z_t = jnp.transpose(z, (1, 0, 2))
    zq_t, idx_t, loss = pl.pallas_call(
        _vq_kernel,
        grid=(_STATE,),
        in_specs=[
            pl.BlockSpec((1, _BATCH, _EMB), lambda s: (s, 0, 0)),
            pl.BlockSpec((1, _DICT, _EMB), lambda s: (s, 0, 0)),
        ],
        out_specs=[
            pl.BlockSpec((1, _BATCH, _EMB), lambda s: (s, 0, 0)),
            pl.BlockSpec((1, 1, _BATCH), lambda s: (s, 0, 0)),
            pl.BlockSpec((1, 1), lambda s: (0, 0)),
        ],
        out_shape=[
            jax.ShapeDtypeStruct((_STATE, _BATCH, _EMB), jnp.float32),
            jax.ShapeDtypeStruct((_STATE, 1, _BATCH), jnp.int32),
            jax.ShapeDtypeStruct((1, 1), jnp.float32),
        ],
        compiler_params=pltpu.CompilerParams(
            dimension_semantics=("arbitrary",)),
    )(z_t, embedding)
    z_q_st = jnp.transpose(zq_t, (1, 0, 2))
    indices = jnp.transpose(idx_t.reshape(_STATE, _BATCH), (1, 0))
    return (z_q_st, loss[0, 0], indices)


# speedup vs baseline: 4.0065x; 1.2233x over previous
import functools

import jax
import jax.numpy as jnp
from jax import lax
from jax.experimental import pallas as pl
from jax.experimental.pallas import tpu as pltpu
from jax.experimental.pallas import tpu_sc as plsc

_STATE = 8
_DICT = 1024
_EMB = 32
_BATCH = 512
_ROWS = _BATCH * _STATE


def _vq_score_kernel(z_ref, emb_ref, idx_ref, gidx_ref, loss_ref):
    s = pl.program_id(0)
    z = z_ref[0]
    e = emb_ref[0]
    en = jnp.sum(e * e, axis=1, keepdims=True)
    z_aug = jnp.concatenate(
        [z, jnp.ones((_BATCH, 1), jnp.float32),
         jnp.zeros((_BATCH, 128 - _EMB - 1), jnp.float32)], axis=1)
    e_aug = jnp.concatenate(
        [-2.0 * e, en,
         jnp.zeros((_DICT, 128 - _EMB - 1), jnp.float32)], axis=1)
    score = jax.lax.dot_general(
        z_aug, e_aug, (((1,), (1,)), ((), ())),
        preferred_element_type=jnp.float32,
        precision=jax.lax.Precision.HIGHEST)
    mins = jnp.min(score, axis=1, keepdims=True)
    kiota = jax.lax.broadcasted_iota(jnp.int32, (_BATCH, _DICT), 1)
    idx = jnp.min(jnp.where(score == mins, kiota, _DICT), axis=1).astype(jnp.int32)
    idx_ref[0, 0] = idx
    gidx_ref[0, 0] = idx + s * _DICT
    zn = jnp.sum(z * z, axis=1, keepdims=True)
    dmin = mins + zn

    @pl.when(s == 0)
    def _init():
        loss_ref[...] = jnp.zeros((1, 1), jnp.float32)

    loss_ref[...] += jnp.sum(dmin).reshape(1, 1)

    @pl.when(s == _STATE - 1)
    def _finalize():
        loss_ref[...] = loss_ref[...] / float(_BATCH * _STATE * _EMB)


_SC_INFO = plsc.get_sparse_core_info()
_NW = _SC_INFO.num_cores * _SC_INFO.num_subcores
_ROWS_PER_W = _ROWS // _NW


@functools.partial(
    pl.kernel,
    mesh=plsc.VectorSubcoreMesh(core_axis_name="c", subcore_axis_name="s"),
    out_type=jax.ShapeDtypeStruct((_ROWS, _EMB), jnp.float32),
    scratch_types=[
        pltpu.VMEM((_ROWS_PER_W,), jnp.int32),
        pltpu.VMEM((_ROWS_PER_W, _EMB), jnp.float32),
        pltpu.SemaphoreType.DMA,
    ],
    compiler_params=pltpu.CompilerParams(use_tc_tiling_on_sc=False),
)
def _sc_gather(table_hbm, gidx_hbm, out_hbm, idx_v, rows_v, sem):
    wid = lax.axis_index("s") * _SC_INFO.num_cores + lax.axis_index("c")
    base = wid * _ROWS_PER_W
    pltpu.sync_copy(gidx_hbm.at[pl.ds(base, _ROWS_PER_W)], idx_v)
    pltpu.async_copy(table_hbm.at[idx_v], rows_v, sem).wait()
    pltpu.sync_copy(rows_v, out_hbm.at[pl.ds(base, _ROWS_PER_W)])


def kernel(z, embedding, ema_cluster_size, ema_w):
    del ema_cluster_size, ema_w
    z_t = jnp.transpose(z, (1, 0, 2))
    idx_t, gidx_t, loss = pl.pallas_call(
        _vq_score_kernel,
        grid=(_STATE,),
        in_specs=[
            pl.BlockSpec((1, _BATCH, _EMB), lambda s: (s, 0, 0)),
            pl.BlockSpec((1, _DICT, _EMB), lambda s: (s, 0, 0)),
        ],
        out_specs=[
            pl.BlockSpec((1, 1, _BATCH), lambda s: (s, 0, 0)),
            pl.BlockSpec((1, 1, _BATCH), lambda s: (s, 0, 0)),
            pl.BlockSpec((1, 1), lambda s: (0, 0)),
        ],
        out_shape=[
            jax.ShapeDtypeStruct((_STATE, 1, _BATCH), jnp.int32),
            jax.ShapeDtypeStruct((_STATE, 1, _BATCH), jnp.int32),
            jax.ShapeDtypeStruct((1, 1), jnp.float32),
        ],
        compiler_params=pltpu.CompilerParams(
            dimension_semantics=("arbitrary",)),
    )(z_t, embedding)
    indices = jnp.transpose(idx_t.reshape(_STATE, _BATCH), (1, 0))
    gidx = jnp.transpose(gidx_t.reshape(_STATE, _BATCH), (1, 0)).reshape(_ROWS)
    table = embedding.reshape(_STATE * _DICT, _EMB)
    z_q_st = _sc_gather(table, gidx).reshape(_BATCH, _STATE, _EMB)
    return (z_q_st, loss[0, 0], indices)


# speedup vs baseline: 4.7995x; 1.1979x over previous
import functools

import jax
import jax.numpy as jnp
from jax import lax
from jax.experimental import pallas as pl
from jax.experimental.pallas import tpu as pltpu
from jax.experimental.pallas import tpu_sc as plsc

_STATE = 8
_DICT = 1024
_EMB = 32
_BATCH = 512
_ROWS = _BATCH * _STATE


def _vq_score_kernel(z_ref, emb_ref, idx_ref, gidx_ref, loss_ref):
    z_all = z_ref[...]
    loss_parts = []
    for s in range(_STATE):
        z = z_all[:, s * _EMB:(s + 1) * _EMB]
        e = emb_ref[s]
        en = jnp.sum(e * e, axis=1, keepdims=True)
        z_aug = jnp.concatenate(
            [z, jnp.ones((_BATCH, 1), jnp.float32),
             jnp.zeros((_BATCH, 128 - _EMB - 1), jnp.float32)], axis=1)
        e_aug = jnp.concatenate(
            [-2.0 * e, en,
             jnp.zeros((_DICT, 128 - _EMB - 1), jnp.float32)], axis=1)
        score = jax.lax.dot_general(
            z_aug, e_aug, (((1,), (1,)), ((), ())),
            preferred_element_type=jnp.float32,
            precision=jax.lax.Precision.HIGHEST)
        mins = jnp.min(score, axis=1, keepdims=True)
        kiota = jax.lax.broadcasted_iota(jnp.int32, (_BATCH, _DICT), 1)
        idx = jnp.min(jnp.where(score == mins, kiota, _DICT), axis=1)
        idx = idx.astype(jnp.int32)
        idx_ref[:, s:s + 1] = idx[:, None]
        gidx_ref[:, s:s + 1] = idx[:, None] + s * _DICT
        zn = jnp.sum(z * z, axis=1, keepdims=True)
        loss_parts.append(jnp.sum(mins + zn))
    total = loss_parts[0]
    for p in loss_parts[1:]:
        total = total + p
    loss_ref[...] = (total / float(_BATCH * _STATE * _EMB)).reshape(1, 1)


_SC_INFO = plsc.get_sparse_core_info()
_NW = _SC_INFO.num_cores * _SC_INFO.num_subcores
_ROWS_PER_W = _ROWS // _NW


@functools.partial(
    pl.kernel,
    mesh=plsc.VectorSubcoreMesh(core_axis_name="c", subcore_axis_name="s"),
    out_type=jax.ShapeDtypeStruct((_ROWS, _EMB), jnp.float32),
    scratch_types=[
        pltpu.VMEM((_ROWS_PER_W,), jnp.int32),
        pltpu.VMEM((_ROWS_PER_W, _EMB), jnp.float32),
        pltpu.SemaphoreType.DMA,
    ],
    compiler_params=pltpu.CompilerParams(use_tc_tiling_on_sc=False),
)
def _sc_gather(table_hbm, gidx_hbm, out_hbm, idx_v, rows_v, sem):
    wid = lax.axis_index("s") * _SC_INFO.num_cores + lax.axis_index("c")
    base = wid * _ROWS_PER_W
    pltpu.sync_copy(gidx_hbm.at[pl.ds(base, _ROWS_PER_W)], idx_v)
    pltpu.async_copy(table_hbm.at[idx_v], rows_v, sem).wait()
    pltpu.sync_copy(rows_v, out_hbm.at[pl.ds(base, _ROWS_PER_W)])


def kernel(z, embedding, ema_cluster_size, ema_w):
    del ema_cluster_size, ema_w
    z_flat = z.reshape(_BATCH, _STATE * _EMB)
    indices, gidx_bs, loss = pl.pallas_call(
        _vq_score_kernel,
        out_shape=[
            jax.ShapeDtypeStruct((_BATCH, _STATE), jnp.int32),
            jax.ShapeDtypeStruct((_BATCH, _STATE), jnp.int32),
            jax.ShapeDtypeStruct((1, 1), jnp.float32),
        ],
    )(z_flat, embedding)
    gidx = gidx_bs.reshape(_ROWS)
    table = embedding.reshape(_STATE * _DICT, _EMB)
    z_q_st = _sc_gather(table, gidx).reshape(_BATCH, _STATE, _EMB)
    return (z_q_st, loss[0, 0], indices)
